# baseline (speedup 1.0000x reference)
import functools
import jax
import jax.numpy as jnp
from jax import lax
from jax.experimental import pallas as pl
from jax.experimental.pallas import tpu as pltpu
from jax.experimental.pallas import tpu_sc as plsc

B, S, D, V = 4096, 200, 128, 100000
NC, NS = 2, 16
NW = NC * NS
RPW = B // NW
NG = (S + 15) // 16
SP = 256
NR = 4
NI = 8
VP = 100352

_mesh = plsc.VectorSubcoreMesh(core_axis_name="c", subcore_axis_name="s")

_scratch = (
    [pltpu.VMEM((RPW, D), jnp.float32)]
    + [pltpu.VMEM((SP,), jnp.int32) for _ in range(NI)]
    + [pltpu.VMEM((S, D), jnp.float32) for _ in range(NR)]
    + [pltpu.VMEM((NG * 16,), jnp.float32) for _ in range(NR)]
    + [pltpu.VMEM((SP,), jnp.float32) for _ in range(2)]
    + [pltpu.VMEM_SHARED((VP,), jnp.float32)]
    + [pltpu.SemaphoreType.DMA for _ in range(NI + 2 * NR + 2)]
)


@functools.partial(
    pl.kernel,
    out_type=jax.ShapeDtypeStruct((B, SP), jnp.float32),
    mesh=_mesh,
    compiler_params=pltpu.CompilerParams(needs_layout_passes=False),
    scratch_types=_scratch,
)
def _sparse_linear(embed, shortlist, weight, bias, out, emb_c, *bufs):
    ivs = bufs[0:NI]
    rvs = bufs[NI:NI + NR]
    bvs = bufs[NI + NR:NI + 2 * NR]
    ovs = bufs[NI + 2 * NR:NI + 2 * NR + 2]
    bsh = bufs[NI + 2 * NR + 2]
    sems = bufs[NI + 2 * NR + 3:]
    sis = sems[0:NI]
    sgs = sems[NI:NI + NR]
    sbs = sems[NI + NR:NI + 2 * NR]
    sos = sems[NI + 2 * NR:]

    wid = lax.axis_index("s") * NC + lax.axis_index("c")
    base = wid * RPW
    iota = lax.iota(jnp.int32, 16)
    sidx = [jnp.minimum(iota + 16 * g, S - 1) for g in range(NG)]

    pltpu.sync_copy(embed.at[pl.ds(base, RPW)], emb_c)
    seg = VP // NS
    sid = lax.axis_index("s")
    pltpu.sync_copy(bias.at[pl.ds(sid * seg, seg)],
                    bsh.at[pl.ds(sid * seg, seg)])
    plsc.subcore_barrier()

    def fire_idx(i, k):
        ii = jnp.minimum(i, RPW - 1)
        pltpu.async_copy(shortlist.at[pl.ds((base + ii) * SP, SP)],
                         ivs[k], sis[k])

    def wait_idx(k):
        pltpu.make_async_copy(shortlist.at[pl.ds(0, SP)], ivs[k],
                              sis[k]).wait()

    def fire_gather(ik, rk):
        idx = ivs[ik].at[pl.ds(0, S)]
        pltpu.async_copy(weight.at[idx], rvs[rk], sgs[rk])
        pltpu.async_copy(bsh.at[idx], bvs[rk].at[pl.ds(0, S)], sbs[rk])

    def wait_gather(rk):
        idx = ivs[0].at[pl.ds(0, S)]
        pltpu.make_async_copy(weight.at[idx], rvs[rk], sgs[rk]).wait()
        pltpu.make_async_copy(bsh.at[idx], bvs[rk].at[pl.ds(0, S)],
                              sbs[rk]).wait()

    def compute_accs(i, rk):
        accs0 = tuple(bvs[rk][pl.ds(16 * g, 16)] for g in range(NG))
        irow = jnp.full((16,), i, jnp.int32)

        def dstep(d, accs):
            dd = (jnp.full((16,), d, jnp.int32) + iota) & (D - 1)
            e = plsc.load_gather(emb_c, [irow, dd])
            return tuple(a + plsc.load_gather(rvs[rk], [sidx[g], dd]) * e
                         for g, a in enumerate(accs))

        return lax.fori_loop(0, D, dstep, accs0, unroll=2)

    for k in range(NI):
        fire_idx(k, k)
    for k in range(NR - 1):
        wait_idx(k)
        fire_gather(k, k)

    @pl.loop(0, RPW // NI)
    def _oct(j):
        for k in range(NI):
            i = NI * j + k
            rk = k % NR
            ok = k % 2

            wait_gather(rk)
            accs = compute_accs(i, rk)
            fire_idx(i + NI, k)
            wait_idx((k + NR - 1) % NI)
            fire_gather((k + NR - 1) % NI, (k + NR - 1) % NR)

            @pl.when(j > 0)
            def _drain():
                pltpu.make_async_copy(ovs[ok], out.at[base], sos[ok]).wait()

            for g in range(NG):
                ovs[ok][pl.ds(16 * g, 16)] = accs[g]
            pltpu.async_copy(ovs[ok], out.at[base + i], sos[ok])

    for ok in range(2):
        pltpu.make_async_copy(ovs[ok], out.at[base], sos[ok]).wait()
    for k in range(NR - 1):
        wait_gather(k)
    for k in range(NR - 1, NI):
        wait_idx(k)


def kernel(embed, shortlist, weight, bias):
    sl = jnp.pad(shortlist.astype(jnp.int32), ((0, 0), (0, SP - S)))
    bp = jnp.pad(bias.reshape(V), (0, VP - V))
    out = _sparse_linear(embed, sl.reshape(B * SP), weight, bp)
    return out[:, :S]


# speedup vs baseline: 1.0073x; 1.0073x over previous
import functools
import jax
import jax.numpy as jnp
from jax import lax
from jax.experimental import pallas as pl
from jax.experimental.pallas import tpu as pltpu
from jax.experimental.pallas import tpu_sc as plsc

B, S, D, V = 4096, 200, 128, 100000
NC, NS = 2, 16
NW = NC * NS
RPW = B // NW
NG = (S + 15) // 16
SP = 256
NR = 4
NI = 8

_mesh = plsc.VectorSubcoreMesh(core_axis_name="c", subcore_axis_name="s")

_scratch = (
    [pltpu.VMEM((RPW, D), jnp.float32)]
    + [pltpu.VMEM((SP,), jnp.int32) for _ in range(NI)]
    + [pltpu.VMEM((S, D), jnp.float32) for _ in range(NR)]
    + [pltpu.VMEM((NG * 16,), jnp.float32) for _ in range(NR)]
    + [pltpu.VMEM((SP,), jnp.float32) for _ in range(2)]
    + [pltpu.SemaphoreType.DMA for _ in range(NI + 2 * NR + 2)]
)


@functools.partial(
    pl.kernel,
    out_type=jax.ShapeDtypeStruct((B, SP), jnp.float32),
    mesh=_mesh,
    compiler_params=pltpu.CompilerParams(needs_layout_passes=False),
    scratch_types=_scratch,
)
def _sparse_linear(embed, shortlist, weight, bias, out, emb_c, *bufs):
    ivs = bufs[0:NI]
    rvs = bufs[NI:NI + NR]
    bvs = bufs[NI + NR:NI + 2 * NR]
    ovs = bufs[NI + 2 * NR:NI + 2 * NR + 2]
    sems = bufs[NI + 2 * NR + 2:]
    sis = sems[0:NI]
    sgs = sems[NI:NI + NR]
    sbs = sems[NI + NR:NI + 2 * NR]
    sos = sems[NI + 2 * NR:]

    wid = lax.axis_index("s") * NC + lax.axis_index("c")
    base = wid * RPW
    iota = lax.iota(jnp.int32, 16)
    sidx = [jnp.minimum(iota + 16 * g, S - 1) for g in range(NG)]

    pltpu.sync_copy(embed.at[pl.ds(base, RPW)], emb_c)

    def fire_idx(i, k):
        ii = jnp.minimum(i, RPW - 1)
        pltpu.async_copy(shortlist.at[pl.ds((base + ii) * SP, SP)],
                         ivs[k], sis[k])

    def wait_idx(k):
        pltpu.make_async_copy(shortlist.at[pl.ds(0, SP)], ivs[k],
                              sis[k]).wait()

    def fire_gather(ik, rk):
        idx = ivs[ik].at[pl.ds(0, S)]
        pltpu.async_copy(bias.at[idx], bvs[rk].at[pl.ds(0, S)], sbs[rk])
        pltpu.async_copy(weight.at[idx], rvs[rk], sgs[rk])

    def wait_gather(rk):
        idx = ivs[0].at[pl.ds(0, S)]
        pltpu.make_async_copy(weight.at[idx], rvs[rk], sgs[rk]).wait()
        pltpu.make_async_copy(bias.at[idx], bvs[rk].at[pl.ds(0, S)],
                              sbs[rk]).wait()

    def compute_accs(i, rk):
        accs0 = tuple(bvs[rk][pl.ds(16 * g, 16)] for g in range(NG))
        irow = jnp.full((16,), i, jnp.int32)

        def dstep(d, accs):
            dd = (jnp.full((16,), d, jnp.int32) + iota) & (D - 1)
            e = plsc.load_gather(emb_c, [irow, dd])
            return tuple(a + plsc.load_gather(rvs[rk], [sidx[g], dd]) * e
                         for g, a in enumerate(accs))

        return lax.fori_loop(0, D, dstep, accs0, unroll=2)

    for k in range(NI):
        fire_idx(k, k)
    for k in range(NR - 1):
        wait_idx(k)
        fire_gather(k, k)

    @pl.loop(0, RPW // NI)
    def _oct(j):
        for k in range(NI):
            i = NI * j + k
            rk = k % NR
            ok = k % 2

            wait_gather(rk)
            accs = compute_accs(i, rk)
            fire_idx(i + NI, k)
            wait_idx((k + NR - 1) % NI)
            fire_gather((k + NR - 1) % NI, (k + NR - 1) % NR)

            @pl.when(j > 0)
            def _drain():
                pltpu.make_async_copy(ovs[ok], out.at[base], sos[ok]).wait()

            for g in range(NG):
                ovs[ok][pl.ds(16 * g, 16)] = accs[g]
            pltpu.async_copy(ovs[ok], out.at[base + i], sos[ok])

    for ok in range(2):
        pltpu.make_async_copy(ovs[ok], out.at[base], sos[ok]).wait()
    for k in range(NR - 1):
        wait_gather(k)
    for k in range(NR - 1, NI):
        wait_idx(k)


def kernel(embed, shortlist, weight, bias):
    sl = jnp.pad(shortlist.astype(jnp.int32), ((0, 0), (0, SP - S)))
    out = _sparse_linear(embed, sl.reshape(B * SP), weight, bias.reshape(V))
    return out[:, :S]
